# baseline (speedup 1.0000x reference)
import functools

import jax
import jax.numpy as jnp
from jax.experimental import pallas as pl
from jax.experimental.pallas import tpu as pltpu

_TEMP = 0.1
_BASE_TEMP = 0.1
_EPS = 1e-12
_INV_T = 10.0
_F8_SCALE = 64.0


def _wsc_kernel(f_ref, labi_ref, labj_ref, sw_ref, labcol_ref, out_ref,
                fn8, sr_acc, sc_acc, g_acc, c_acc, *, b, ni, na, bn, nt, cpad):
    t = pl.program_id(0)

    @pl.when(t == 0)
    def _init():
        sr_acc[...] = jnp.zeros_like(sr_acc)
        sc_acc[...] = jnp.zeros_like(sc_acc)
        g_acc[...] = jnp.zeros_like(g_acc)
        c_acc[...] = jnp.zeros_like(c_acc)

    @pl.when(t < na)
    def _normalize():
        f = f_ref[...]
        r = jax.lax.rsqrt(jnp.maximum(jnp.sum(f * f, axis=1, keepdims=True),
                                      1e-24))
        fn8[pl.ds(t * bn, bn), :] = (f * (r * _F8_SCALE)).astype(
            jnp.float8_e4m3fn)

    @pl.when(t >= na)
    def _main():
        tc = t - na
        i = tc % ni
        off = tc // ni
        j = (i + off) % ni

        fi = fn8[pl.ds(i * b, b), :]
        fj = fn8[pl.ds(j * b, b), :]
        sim = jax.lax.dot_general(fi, fj, (((1,), (1,)), ((), ())),
                                  preferred_element_type=jnp.float32)
        sim = sim * (_INV_T / (_F8_SCALE * _F8_SCALE))

        labj = labj_ref[...]
        ohj = (labj == jax.lax.broadcasted_iota(jnp.int32, (cpad, b), 0)
               ).astype(jnp.bfloat16)

        @pl.when(off == 0)
        def _diag_block():
            offd = (jax.lax.broadcasted_iota(jnp.int32, (b, b), 0)
                    != jax.lax.broadcasted_iota(jnp.int32, (b, b), 1))
            e = jnp.where(offd, jnp.exp(sim - _INV_T), 0.0)
            sr_acc[pl.ds(i * b, b), :] += jnp.sum(e, axis=1, keepdims=True)
            simz = jnp.where(offd, sim, 0.0)
            g_acc[pl.ds(i * b, b), :] += jax.lax.dot_general(
                simz.astype(jnp.bfloat16), ohj, (((1,), (1,)), ((), ())),
                preferred_element_type=jnp.float32)
            c_acc[...] += jnp.sum(ohj.astype(jnp.float32), axis=1,
                                  keepdims=True)

        @pl.when(off != 0)
        def _offdiag_block():
            e = jnp.exp(sim - _INV_T)
            sr_acc[pl.ds(i * b, b), :] += jnp.sum(e, axis=1, keepdims=True)
            g_acc[pl.ds(i * b, b), :] += jax.lax.dot_general(
                sim.astype(jnp.bfloat16), ohj, (((1,), (1,)), ((), ())),
                preferred_element_type=jnp.float32)

            @pl.when(off < ni // 2)
            def _col_side():
                sc_acc[:, pl.ds(j * b, b)] += jnp.sum(e, axis=0, keepdims=True)
                labi = labi_ref[...]
                ohi = (labi == jax.lax.broadcasted_iota(jnp.int32, (cpad, b), 0)
                       ).astype(jnp.bfloat16)
                g_acc[pl.ds(j * b, b), :] += jax.lax.dot_general(
                    sim.astype(jnp.bfloat16), ohi, (((0,), (1,)), ((), ())),
                    preferred_element_type=jnp.float32)

    @pl.when(t == na + nt - 1)
    def _emit():
        B = ni * b
        S = sr_acc[...] + jnp.transpose(sc_acc[...])
        sw = sw_ref[...]
        ohi = (labcol_ref[...] == jax.lax.broadcasted_iota(
            jnp.int32, (B, cpad), 1)).astype(jnp.float32)
        sw_il = jnp.sum(sw * ohi, axis=1, keepdims=True)
        W = jnp.dot(sw, c_acc[...], preferred_element_type=jnp.float32) - sw_il
        P = jnp.sum(sw * g_acc[...], axis=1, keepdims=True)
        logden = _INV_T + jnp.log(S + _EPS)
        out_ref[...] = -(_TEMP / _BASE_TEMP) * (P - W * logden) / (W + _EPS)


@jax.jit
def kernel(features, labels, similarity_weights):
    B, D = features.shape
    C = similarity_weights.shape[1]
    cpad = 128
    b = 512
    bn = 1024
    ni = B // b
    na = B // bn
    nt = ni * (ni // 2 + 1)

    lab32 = labels.astype(jnp.int32)
    labrow = lab32.reshape(1, B)
    labcol = lab32.reshape(B, 1)
    swp = jnp.zeros((B, cpad), jnp.float32).at[:, :C].set(similarity_weights)

    def _i_map(t):
        tc = jnp.maximum(t - na, 0)
        return (0, tc % ni)

    def _j_map(t):
        tc = jnp.maximum(t - na, 0)
        return (0, (tc % ni + tc // ni) % ni)

    out = pl.pallas_call(
        functools.partial(_wsc_kernel, b=b, ni=ni, na=na, bn=bn, nt=nt,
                          cpad=cpad),
        grid=(na + nt,),
        in_specs=[
            pl.BlockSpec((bn, D), lambda t: (jnp.minimum(t, na - 1), 0)),
            pl.BlockSpec((1, b), _i_map),
            pl.BlockSpec((1, b), _j_map),
            pl.BlockSpec((B, cpad), lambda t: (0, 0)),
            pl.BlockSpec((B, 1), lambda t: (0, 0)),
        ],
        out_specs=pl.BlockSpec((B, 1), lambda t: (0, 0)),
        out_shape=jax.ShapeDtypeStruct((B, 1), jnp.float32),
        scratch_shapes=[
            pltpu.VMEM((B, D), jnp.float8_e4m3fn),
            pltpu.VMEM((B, 1), jnp.float32),
            pltpu.VMEM((1, B), jnp.float32),
            pltpu.VMEM((B, cpad), jnp.float32),
            pltpu.VMEM((cpad, 1), jnp.float32),
        ],
        compiler_params=pltpu.CompilerParams(
            dimension_semantics=("arbitrary",)),
    )(features, labrow, labrow, swp, labcol)
    return jnp.mean(out)


# speedup vs baseline: 1.4828x; 1.4828x over previous
import functools

import jax
import jax.numpy as jnp
from jax.experimental import pallas as pl
from jax.experimental.pallas import tpu as pltpu

_TEMP = 0.1
_BASE_TEMP = 0.1
_EPS = 1e-12
_INV_T = 10.0
_F8_SCALE = 64.0
_SIM_SCALE = _INV_T / (_F8_SCALE * _F8_SCALE)


def _wsc_kernel(f_ref, labi_ref, labj_ref, labblk_ref, sw_ref, labcol_ref,
                out_ref, fn8, sr_acc, sc_acc, g_acc, c_acc,
                *, b, ni, nt, cpad):
    t = pl.program_id(0)

    @pl.when(t == 0)
    def _init():
        sr_acc[...] = jnp.zeros_like(sr_acc)
        sc_acc[...] = jnp.zeros_like(sc_acc)
        g_acc[...] = jnp.zeros_like(g_acc)
        c_acc[...] = jnp.zeros_like(c_acc)

    @pl.when(t < ni)
    def _normalize_and_diag():
        f = f_ref[...]
        r = jax.lax.rsqrt(jnp.maximum(jnp.sum(f * f, axis=1, keepdims=True),
                                      1e-24))
        q8 = (f * (r * _F8_SCALE)).astype(jnp.float8_e4m3fn)
        fn8[pl.ds(t * b, b), :] = q8

        qf = q8.astype(jnp.float32)
        simdiag = jnp.sum(qf * qf, axis=1, keepdims=True) * _SIM_SCALE
        sim = jax.lax.dot_general(q8, q8, (((1,), (1,)), ((), ())),
                                  preferred_element_type=jnp.float32)
        sim = sim * _SIM_SCALE
        e = jnp.exp(sim - _INV_T)
        sr_acc[pl.ds(t * b, b), :] += (jnp.sum(e, axis=1, keepdims=True)
                                       - jnp.exp(simdiag - _INV_T))
        labj = labj_ref[...]
        ohj = (labj == jax.lax.broadcasted_iota(jnp.int32, (cpad, b), 0)
               ).astype(jnp.bfloat16)
        ohi_loc = (labblk_ref[...] == jax.lax.broadcasted_iota(
            jnp.int32, (b, cpad), 1)).astype(jnp.float32)
        g_acc[pl.ds(t * b, b), :] += (
            jax.lax.dot_general(sim.astype(jnp.bfloat16), ohj,
                                (((1,), (1,)), ((), ())),
                                preferred_element_type=jnp.float32)
            - ohi_loc * simdiag)
        c_acc[...] += jnp.sum(ohj.astype(jnp.float32), axis=1, keepdims=True)

    @pl.when(t >= ni)
    def _offdiag():
        tc = t - ni
        i = tc % ni
        off = tc // ni + 1
        j = (i + off) % ni

        fi = fn8[pl.ds(i * b, b), :]
        fj = fn8[pl.ds(j * b, b), :]
        sim = jax.lax.dot_general(fi, fj, (((1,), (1,)), ((), ())),
                                  preferred_element_type=jnp.float32)
        sim = sim * _SIM_SCALE

        labj = labj_ref[...]
        ohj = (labj == jax.lax.broadcasted_iota(jnp.int32, (cpad, b), 0)
               ).astype(jnp.bfloat16)

        e = jnp.exp(sim - _INV_T)
        sr_acc[pl.ds(i * b, b), :] += jnp.sum(e, axis=1, keepdims=True)
        g_acc[pl.ds(i * b, b), :] += jax.lax.dot_general(
            sim.astype(jnp.bfloat16), ohj, (((1,), (1,)), ((), ())),
            preferred_element_type=jnp.float32)

        @pl.when(off < ni // 2)
        def _col_side():
            sc_acc[:, pl.ds(j * b, b)] += jnp.sum(e, axis=0, keepdims=True)
            labi = labi_ref[...]
            ohi = (labi == jax.lax.broadcasted_iota(jnp.int32, (cpad, b), 0)
                   ).astype(jnp.bfloat16)
            g_acc[pl.ds(j * b, b), :] += jax.lax.dot_general(
                sim.astype(jnp.bfloat16), ohi, (((0,), (1,)), ((), ())),
                preferred_element_type=jnp.float32)

    @pl.when(t == nt - 1)
    def _emit():
        B = ni * b
        S = sr_acc[...] + jnp.transpose(sc_acc[...])
        sw = sw_ref[...]
        ohi = (labcol_ref[...] == jax.lax.broadcasted_iota(
            jnp.int32, (B, cpad), 1)).astype(jnp.float32)
        sw_il = jnp.sum(sw * ohi, axis=1, keepdims=True)
        W = jnp.dot(sw, c_acc[...], preferred_element_type=jnp.float32) - sw_il
        P = jnp.sum(sw * g_acc[...], axis=1, keepdims=True)
        logden = _INV_T + jnp.log(S + _EPS)
        loss = -(_TEMP / _BASE_TEMP) * (P - W * logden) / (W + _EPS)
        out_ref[...] = jnp.sum(loss, keepdims=True).reshape(1, 1) * (1.0 / B)


@jax.jit
def kernel(features, labels, similarity_weights):
    B, D = features.shape
    C = similarity_weights.shape[1]
    cpad = 128
    b = 1024
    ni = B // b
    nt = ni + ni * (ni // 2)

    lab32 = labels.astype(jnp.int32)
    labrow = lab32.reshape(1, B)
    labcol = lab32.reshape(B, 1)
    swp = jnp.zeros((B, cpad), jnp.float32).at[:, :C].set(similarity_weights)

    def _i_map(t):
        tc = jnp.maximum(t - ni, 0)
        return (0, jnp.where(t < ni, t, tc % ni))

    def _j_map(t):
        tc = jnp.maximum(t - ni, 0)
        return (0, jnp.where(t < ni, t, (tc % ni + tc // ni + 1) % ni))

    out = pl.pallas_call(
        functools.partial(_wsc_kernel, b=b, ni=ni, nt=nt, cpad=cpad),
        grid=(nt,),
        in_specs=[
            pl.BlockSpec((b, D), lambda t: (jnp.minimum(t, ni - 1), 0)),
            pl.BlockSpec((1, b), _i_map),
            pl.BlockSpec((1, b), _j_map),
            pl.BlockSpec((b, 1), lambda t: (jnp.minimum(t, ni - 1), 0)),
            pl.BlockSpec((B, cpad), lambda t: (0, 0)),
            pl.BlockSpec((B, 1), lambda t: (0, 0)),
        ],
        out_specs=pl.BlockSpec((1, 1), lambda t: (0, 0)),
        out_shape=jax.ShapeDtypeStruct((1, 1), jnp.float32),
        scratch_shapes=[
            pltpu.VMEM((B, D), jnp.float8_e4m3fn),
            pltpu.VMEM((B, 1), jnp.float32),
            pltpu.VMEM((1, B), jnp.float32),
            pltpu.VMEM((B, cpad), jnp.float32),
            pltpu.VMEM((cpad, 1), jnp.float32),
        ],
        compiler_params=pltpu.CompilerParams(
            dimension_semantics=("arbitrary",)),
    )(features, labrow, labrow, labcol, swp, labcol)
    return out[0, 0]


# speedup vs baseline: 1.5358x; 1.0358x over previous
import functools

import jax
import jax.numpy as jnp
from jax.experimental import pallas as pl
from jax.experimental.pallas import tpu as pltpu

_TEMP = 0.1
_BASE_TEMP = 0.1
_EPS = 1e-12
_INV_T = 10.0
_F8_SCALE = 64.0
_SIM_SCALE = _INV_T / (_F8_SCALE * _F8_SCALE)


def _wsc_kernel(f_ref, labi_ref, labj_ref, sw_ref, labcol_ref,
                out_ref, fn8, sr_acc, sc_acc, g_acc, c_acc,
                *, b, ni, nt, cpad):
    t = pl.program_id(0)

    @pl.when(t == 0)
    def _init():
        sr_acc[...] = jnp.zeros_like(sr_acc)
        sc_acc[...] = jnp.zeros_like(sc_acc)
        g_acc[...] = jnp.zeros_like(g_acc)
        c_acc[...] = jnp.zeros_like(c_acc)

    @pl.when(t < ni)
    def _normalize_and_diag():
        f = f_ref[...]
        r = jax.lax.rsqrt(jnp.maximum(jnp.sum(f * f, axis=1, keepdims=True),
                                      1e-24))
        q8 = (f * (r * _F8_SCALE)).astype(jnp.float8_e4m3fn)
        fn8[pl.ds(t * b, b), :] = q8

        sim = jax.lax.dot_general(q8, q8, (((1,), (1,)), ((), ())),
                                  preferred_element_type=jnp.float32)
        sim = sim * _SIM_SCALE
        ond = (jax.lax.broadcasted_iota(jnp.int32, (b, b), 0)
               == jax.lax.broadcasted_iota(jnp.int32, (b, b), 1))
        e = jnp.where(ond, 0.0, jnp.exp(sim - _INV_T))
        sr_acc[pl.ds(t * b, b), :] += jnp.sum(e, axis=1, keepdims=True)
        labj = labj_ref[...]
        ohj = (labj == jax.lax.broadcasted_iota(jnp.int32, (cpad, b), 0)
               ).astype(jnp.bfloat16)
        simz = jnp.where(ond, 0.0, sim)
        g_acc[pl.ds(t * b, b), :] += jax.lax.dot_general(
            simz.astype(jnp.bfloat16), ohj, (((1,), (1,)), ((), ())),
            preferred_element_type=jnp.float32)
        c_acc[...] += jnp.sum(ohj.astype(jnp.float32), axis=1, keepdims=True)

    @pl.when(t >= ni)
    def _offdiag():
        tc = t - ni
        i = tc % ni
        off = tc // ni + 1
        j = (i + off) % ni

        fi = fn8[pl.ds(i * b, b), :]
        fj = fn8[pl.ds(j * b, b), :]
        sim = jax.lax.dot_general(fi, fj, (((1,), (1,)), ((), ())),
                                  preferred_element_type=jnp.float32)
        sim = sim * _SIM_SCALE

        labj = labj_ref[...]
        ohj = (labj == jax.lax.broadcasted_iota(jnp.int32, (cpad, b), 0)
               ).astype(jnp.bfloat16)

        e = jnp.exp(sim - _INV_T)
        sr_acc[pl.ds(i * b, b), :] += jnp.sum(e, axis=1, keepdims=True)
        g_acc[pl.ds(i * b, b), :] += jax.lax.dot_general(
            sim.astype(jnp.bfloat16), ohj, (((1,), (1,)), ((), ())),
            preferred_element_type=jnp.float32)

        @pl.when(off < ni // 2)
        def _col_side():
            sc_acc[:, pl.ds(j * b, b)] += jnp.sum(e, axis=0, keepdims=True)
            labi = labi_ref[...]
            ohi = (labi == jax.lax.broadcasted_iota(jnp.int32, (cpad, b), 0)
                   ).astype(jnp.bfloat16)
            g_acc[pl.ds(j * b, b), :] += jax.lax.dot_general(
                sim.astype(jnp.bfloat16), ohi, (((0,), (1,)), ((), ())),
                preferred_element_type=jnp.float32)

    @pl.when(t == nt - 1)
    def _emit():
        B = ni * b
        S = sr_acc[...] + jnp.transpose(sc_acc[...])
        sw = sw_ref[...]
        ohi = (labcol_ref[...] == jax.lax.broadcasted_iota(
            jnp.int32, (B, cpad), 1)).astype(jnp.float32)
        sw_il = jnp.sum(sw * ohi, axis=1, keepdims=True)
        W = jnp.dot(sw, c_acc[...], preferred_element_type=jnp.float32) - sw_il
        P = jnp.sum(sw * g_acc[...], axis=1, keepdims=True)
        logden = _INV_T + jnp.log(S + _EPS)
        loss = -(_TEMP / _BASE_TEMP) * (P - W * logden) / (W + _EPS)
        out_ref[...] = jnp.sum(loss, keepdims=True).reshape(1, 1) * (1.0 / B)


@jax.jit
def kernel(features, labels, similarity_weights):
    B, D = features.shape
    C = similarity_weights.shape[1]
    cpad = 128
    b = 1024
    ni = B // b
    nt = ni + ni * (ni // 2)

    lab32 = labels.astype(jnp.int32)
    labrow = lab32.reshape(1, B)
    labcol = lab32.reshape(B, 1)
    swp = jnp.zeros((B, cpad), jnp.float32).at[:, :C].set(similarity_weights)

    def _i_map(t):
        tc = jnp.maximum(t - ni, 0)
        return (0, jnp.where(t < ni, t, tc % ni))

    def _j_map(t):
        tc = jnp.maximum(t - ni, 0)
        return (0, jnp.where(t < ni, t, (tc % ni + tc // ni + 1) % ni))

    out = pl.pallas_call(
        functools.partial(_wsc_kernel, b=b, ni=ni, nt=nt, cpad=cpad),
        grid=(nt,),
        in_specs=[
            pl.BlockSpec((b, D), lambda t: (jnp.minimum(t, ni - 1), 0)),
            pl.BlockSpec((1, b), _i_map),
            pl.BlockSpec((1, b), _j_map),
            pl.BlockSpec((B, cpad), lambda t: (0, 0)),
            pl.BlockSpec((B, 1), lambda t: (0, 0)),
        ],
        out_specs=pl.BlockSpec((1, 1), lambda t: (0, 0)),
        out_shape=jax.ShapeDtypeStruct((1, 1), jnp.float32),
        scratch_shapes=[
            pltpu.VMEM((B, D), jnp.float8_e4m3fn),
            pltpu.VMEM((B, 1), jnp.float32),
            pltpu.VMEM((1, B), jnp.float32),
            pltpu.VMEM((B, cpad), jnp.float32),
            pltpu.VMEM((cpad, 1), jnp.float32),
        ],
        compiler_params=pltpu.CompilerParams(
            dimension_semantics=("arbitrary",)),
    )(features, labrow, labrow, swp, labcol)
    return out[0, 0]


# speedup vs baseline: 1.6097x; 1.0481x over previous
import functools
import math

import jax
import jax.numpy as jnp
from jax.experimental import pallas as pl
from jax.experimental.pallas import tpu as pltpu

_TEMP = 0.1
_BASE_TEMP = 0.1
_EPS = 1e-12
_INV_T = 10.0
_F8_SCALE = 64.0
_SIM_SCALE = _INV_T / (_F8_SCALE * _F8_SCALE)
_EXP_MUL = _SIM_SCALE * math.log2(math.e)
_EXP_OFF = _INV_T * math.log2(math.e)


def _wsc_kernel(f_ref, labi_ref, labj_ref, sw_ref, labcol_ref,
                out_ref, fn8, sr_acc, sc_acc, g_acc, c_acc,
                *, b, ni, nt, C):
    t = pl.program_id(0)

    @pl.when(t == 0)
    def _init():
        sr_acc[...] = jnp.zeros_like(sr_acc)
        sc_acc[...] = jnp.zeros_like(sc_acc)
        g_acc[...] = jnp.zeros_like(g_acc)
        c_acc[...] = jnp.zeros_like(c_acc)

    @pl.when(t < ni)
    def _normalize_and_diag():
        f = f_ref[...]
        r = jax.lax.rsqrt(jnp.maximum(jnp.sum(f * f, axis=1, keepdims=True),
                                      1e-24))
        q8 = (f * (r * _F8_SCALE)).astype(jnp.float8_e4m3fn)
        fn8[pl.ds(t * b, b), :] = q8

        raw = jax.lax.dot_general(q8, q8, (((1,), (1,)), ((), ())),
                                  preferred_element_type=jnp.float32)
        ond = (jax.lax.broadcasted_iota(jnp.int32, (b, b), 0)
               == jax.lax.broadcasted_iota(jnp.int32, (b, b), 1))
        e = jnp.where(ond, 0.0, jnp.exp2(raw * _EXP_MUL - _EXP_OFF))
        sr_acc[pl.ds(t * b, b), :] += jnp.sum(e, axis=1, keepdims=True)
        labj = labj_ref[...]
        ohj = (labj == jax.lax.broadcasted_iota(jnp.int32, (C, b), 0)
               ).astype(jnp.bfloat16)
        rawz = jnp.where(ond, 0.0, raw)
        g_acc[pl.ds(t * b, b), :] += jax.lax.dot_general(
            rawz.astype(jnp.bfloat16), ohj, (((1,), (1,)), ((), ())),
            preferred_element_type=jnp.float32)
        c_acc[...] += jnp.sum(ohj.astype(jnp.float32), axis=1, keepdims=True)

    @pl.when(t >= ni)
    def _offdiag():
        tc = t - ni
        i = tc % ni
        off = tc // ni + 1
        j = (i + off) % ni

        fi = fn8[pl.ds(i * b, b), :]
        fj = fn8[pl.ds(j * b, b), :]
        raw = jax.lax.dot_general(fi, fj, (((1,), (1,)), ((), ())),
                                  preferred_element_type=jnp.float32)

        labj = labj_ref[...]
        ohj = (labj == jax.lax.broadcasted_iota(jnp.int32, (C, b), 0)
               ).astype(jnp.bfloat16)

        e = jnp.exp2(raw * _EXP_MUL - _EXP_OFF)
        sr_acc[pl.ds(i * b, b), :] += jnp.sum(e, axis=1, keepdims=True)
        g_acc[pl.ds(i * b, b), :] += jax.lax.dot_general(
            raw.astype(jnp.bfloat16), ohj, (((1,), (1,)), ((), ())),
            preferred_element_type=jnp.float32)

        @pl.when(off < ni // 2)
        def _col_side():
            sc_acc[:, pl.ds(j * b, b)] += jnp.sum(e, axis=0, keepdims=True)
            labi = labi_ref[...]
            ohi = (labi == jax.lax.broadcasted_iota(jnp.int32, (C, b), 0)
                   ).astype(jnp.bfloat16)
            g_acc[pl.ds(j * b, b), :] += jax.lax.dot_general(
                raw.astype(jnp.bfloat16), ohi, (((0,), (1,)), ((), ())),
                preferred_element_type=jnp.float32)

    @pl.when(t == nt - 1)
    def _emit():
        B = ni * b
        S = sr_acc[...] + jnp.transpose(sc_acc[...])
        sw = sw_ref[...]
        ohi = (labcol_ref[...] == jax.lax.broadcasted_iota(
            jnp.int32, (B, C), 1)).astype(jnp.float32)
        sw_il = jnp.sum(sw * ohi, axis=1, keepdims=True)
        W = jnp.dot(sw, c_acc[...], preferred_element_type=jnp.float32) - sw_il
        P = jnp.sum(sw * g_acc[...], axis=1, keepdims=True) * _SIM_SCALE
        logden = _INV_T + jnp.log(S + _EPS)
        loss = -(_TEMP / _BASE_TEMP) * (P - W * logden) / (W + _EPS)
        out_ref[...] = jnp.sum(loss, keepdims=True).reshape(1, 1) * (1.0 / B)


@jax.jit
def kernel(features, labels, similarity_weights):
    B, D = features.shape
    C = similarity_weights.shape[1]
    b = 1024
    ni = B // b
    nt = ni + ni * (ni // 2)

    lab32 = labels.astype(jnp.int32)
    labrow = lab32.reshape(1, B)
    labcol = lab32.reshape(B, 1)

    def _i_map(t):
        tc = jnp.maximum(t - ni, 0)
        return (0, jnp.where(t < ni, t, tc % ni))

    def _j_map(t):
        tc = jnp.maximum(t - ni, 0)
        return (0, jnp.where(t < ni, t, (tc % ni + tc // ni + 1) % ni))

    out = pl.pallas_call(
        functools.partial(_wsc_kernel, b=b, ni=ni, nt=nt, C=C),
        grid=(nt,),
        in_specs=[
            pl.BlockSpec((b, D), lambda t: (jnp.minimum(t, ni - 1), 0)),
            pl.BlockSpec((1, b), _i_map),
            pl.BlockSpec((1, b), _j_map),
            pl.BlockSpec((B, C), lambda t: (0, 0)),
            pl.BlockSpec((B, 1), lambda t: (0, 0)),
        ],
        out_specs=pl.BlockSpec((1, 1), lambda t: (0, 0)),
        out_shape=jax.ShapeDtypeStruct((1, 1), jnp.float32),
        scratch_shapes=[
            pltpu.VMEM((B, D), jnp.float8_e4m3fn),
            pltpu.VMEM((B, 1), jnp.float32),
            pltpu.VMEM((1, B), jnp.float32),
            pltpu.VMEM((B, C), jnp.float32),
            pltpu.VMEM((C, 1), jnp.float32),
        ],
        compiler_params=pltpu.CompilerParams(
            dimension_semantics=("arbitrary",)),
    )(features, labrow, labrow, similarity_weights, labcol)
    return out[0, 0]


# speedup vs baseline: 1.6834x; 1.0458x over previous
import functools
import math

import jax
import jax.numpy as jnp
from jax.experimental import pallas as pl
from jax.experimental.pallas import tpu as pltpu

_TEMP = 0.1
_BASE_TEMP = 0.1
_EPS = 1e-12
_INV_T = 10.0
_F8_SCALE = 64.0
_SIM_SCALE = _INV_T / (_F8_SCALE * _F8_SCALE)
_EXP_MUL = _SIM_SCALE * math.log2(math.e)
_EXP_OFF = _INV_T * math.log2(math.e)


def _wsc_kernel(f_ref, labi_ref, labj_ref, sw_ref, labcol_ref,
                out_ref, fn8, sr_acc, sc_acc, g_acc, c_acc,
                *, b, ni, nt, C):
    t = pl.program_id(0)

    @pl.when(t == 0)
    def _init():
        sr_acc[...] = jnp.zeros_like(sr_acc)
        sc_acc[...] = jnp.zeros_like(sc_acc)
        g_acc[...] = jnp.zeros_like(g_acc)
        c_acc[...] = jnp.zeros_like(c_acc)

    @pl.when(t < ni)
    def _normalize_and_diag():
        f = f_ref[...]
        r = jax.lax.rsqrt(jnp.maximum(jnp.sum(f * f, axis=1, keepdims=True),
                                      1e-24))
        q8 = (f * (r * _F8_SCALE)).astype(jnp.float8_e4m3fn)
        fn8[pl.ds(t * b, b), :] = q8

        raw = jax.lax.dot_general(q8, q8, (((1,), (1,)), ((), ())),
                                  preferred_element_type=jnp.float32)
        ond = (jax.lax.broadcasted_iota(jnp.int32, (b, b), 0)
               == jax.lax.broadcasted_iota(jnp.int32, (b, b), 1))
        e = jnp.where(ond, 0.0, jnp.exp2(raw * _EXP_MUL - _EXP_OFF))
        sr_acc[pl.ds(t * b, b), :] += jnp.sum(e, axis=1, keepdims=True)
        labj = labj_ref[...]
        ohj = (labj == jax.lax.broadcasted_iota(jnp.int32, (C, b), 0)
               ).astype(jnp.float8_e4m3fn)
        simz = jnp.where(ond, 0.0, raw * _SIM_SCALE)
        g_acc[pl.ds(t * b, b), :] += jax.lax.dot_general(
            simz.astype(jnp.float8_e4m3fn), ohj, (((1,), (1,)), ((), ())),
            preferred_element_type=jnp.float32)
        c_acc[...] += jnp.sum(ohj.astype(jnp.float32), axis=1, keepdims=True)

    @pl.when(t >= ni)
    def _offdiag():
        tc = t - ni
        i = tc % ni
        off = tc // ni + 1
        j = (i + off) % ni

        fi = fn8[pl.ds(i * b, b), :]
        fj = fn8[pl.ds(j * b, b), :]
        raw = jax.lax.dot_general(fi, fj, (((1,), (1,)), ((), ())),
                                  preferred_element_type=jnp.float32)

        labj = labj_ref[...]
        ohj = (labj == jax.lax.broadcasted_iota(jnp.int32, (C, b), 0)
               ).astype(jnp.float8_e4m3fn)

        e = jnp.exp2(raw * _EXP_MUL - _EXP_OFF)
        sim8 = (raw * _SIM_SCALE).astype(jnp.float8_e4m3fn)
        sr_acc[pl.ds(i * b, b), :] += jnp.sum(e, axis=1, keepdims=True)
        g_acc[pl.ds(i * b, b), :] += jax.lax.dot_general(
            sim8, ohj, (((1,), (1,)), ((), ())),
            preferred_element_type=jnp.float32)

        @pl.when(off < ni // 2)
        def _col_side():
            sc_acc[:, pl.ds(j * b, b)] += jnp.sum(e, axis=0, keepdims=True)
            labi = labi_ref[...]
            ohi = (labi == jax.lax.broadcasted_iota(jnp.int32, (C, b), 0)
                   ).astype(jnp.float8_e4m3fn)
            g_acc[pl.ds(j * b, b), :] += jax.lax.dot_general(
                sim8, ohi, (((0,), (1,)), ((), ())),
                preferred_element_type=jnp.float32)

    @pl.when(t == nt - 1)
    def _emit():
        B = ni * b
        S = sr_acc[...] + jnp.transpose(sc_acc[...])
        sw = sw_ref[...]
        ohi = (labcol_ref[...] == jax.lax.broadcasted_iota(
            jnp.int32, (B, C), 1)).astype(jnp.float32)
        sw_il = jnp.sum(sw * ohi, axis=1, keepdims=True)
        W = jnp.dot(sw, c_acc[...], preferred_element_type=jnp.float32) - sw_il
        P = jnp.sum(sw * g_acc[...], axis=1, keepdims=True)
        logden = _INV_T + jnp.log(S + _EPS)
        loss = -(_TEMP / _BASE_TEMP) * (P - W * logden) / (W + _EPS)
        out_ref[...] = jnp.sum(loss, keepdims=True).reshape(1, 1) * (1.0 / B)


@jax.jit
def kernel(features, labels, similarity_weights):
    B, D = features.shape
    C = similarity_weights.shape[1]
    b = 1024
    ni = B // b
    nt = ni + ni * (ni // 2)

    lab32 = labels.astype(jnp.int32)
    labrow = lab32.reshape(1, B)
    labcol = lab32.reshape(B, 1)

    def _i_map(t):
        tc = jnp.maximum(t - ni, 0)
        return (0, jnp.where(t < ni, t, tc % ni))

    def _j_map(t):
        tc = jnp.maximum(t - ni, 0)
        return (0, jnp.where(t < ni, t, (tc % ni + tc // ni + 1) % ni))

    out = pl.pallas_call(
        functools.partial(_wsc_kernel, b=b, ni=ni, nt=nt, C=C),
        grid=(nt,),
        in_specs=[
            pl.BlockSpec((b, D), lambda t: (jnp.minimum(t, ni - 1), 0)),
            pl.BlockSpec((1, b), _i_map),
            pl.BlockSpec((1, b), _j_map),
            pl.BlockSpec((B, C), lambda t: (0, 0)),
            pl.BlockSpec((B, 1), lambda t: (0, 0)),
        ],
        out_specs=pl.BlockSpec((1, 1), lambda t: (0, 0)),
        out_shape=jax.ShapeDtypeStruct((1, 1), jnp.float32),
        scratch_shapes=[
            pltpu.VMEM((B, D), jnp.float8_e4m3fn),
            pltpu.VMEM((B, 1), jnp.float32),
            pltpu.VMEM((1, B), jnp.float32),
            pltpu.VMEM((B, C), jnp.float32),
            pltpu.VMEM((C, 1), jnp.float32),
        ],
        compiler_params=pltpu.CompilerParams(
            dimension_semantics=("arbitrary",)),
    )(features, labrow, labrow, similarity_weights, labcol)
    return out[0, 0]


# speedup vs baseline: 1.9601x; 1.1643x over previous
import functools
import math

import jax
import jax.numpy as jnp
from jax.experimental import pallas as pl
from jax.experimental.pallas import tpu as pltpu

_TEMP = 0.1
_BASE_TEMP = 0.1
_EPS = 1e-12
_INV_T = 10.0
_F8_SCALE = 64.0
_SIM_SCALE = _INV_T / (_F8_SCALE * _F8_SCALE)
_EXP_MUL = _SIM_SCALE * math.log2(math.e)
_EXP_OFF = _INV_T * math.log2(math.e)


def _wsc_kernel(f_ref, labi_ref, labj_ref, sw_ref, labcol_ref,
                out_ref, fn8, sr_acc, sc_acc, g_acc, c_acc,
                *, b, ni, nt, C):
    t = pl.program_id(0)

    @pl.when(t == 0)
    def _init():
        sr_acc[...] = jnp.zeros_like(sr_acc)
        sc_acc[...] = jnp.zeros_like(sc_acc)
        g_acc[...] = jnp.zeros_like(g_acc)
        c_acc[...] = jnp.zeros_like(c_acc)

    @pl.when(t < ni)
    def _normalize_and_diag():
        f = f_ref[...]
        r = jax.lax.rsqrt(jnp.maximum(jnp.sum(f * f, axis=1, keepdims=True),
                                      1e-24))
        q8 = (f * (r * _F8_SCALE)).astype(jnp.float8_e4m3fn)
        fn8[pl.ds(t * b, b), :] = q8

        raw = jax.lax.dot_general(q8, q8, (((1,), (1,)), ((), ())),
                                  preferred_element_type=jnp.float32)
        ond = (jax.lax.broadcasted_iota(jnp.int32, (b, b), 0)
               == jax.lax.broadcasted_iota(jnp.int32, (b, b), 1))
        e = jnp.where(ond, 0.0, jnp.exp2(raw * _EXP_MUL - _EXP_OFF))
        sr_acc[pl.ds(t * b, b), :] += jnp.sum(e, axis=1, keepdims=True)
        labj = labj_ref[...]
        ohj = (labj == jax.lax.broadcasted_iota(jnp.int32, (C, b), 0)
               ).astype(jnp.float8_e4m3fn)
        simz = jnp.where(ond, 0.0, raw * _SIM_SCALE)
        g_acc[pl.ds(t * b, b), :] += jax.lax.dot_general(
            simz.astype(jnp.float8_e4m3fn), ohj, (((1,), (1,)), ((), ())),
            preferred_element_type=jnp.float32)
        c_acc[...] += jnp.sum(ohj.astype(jnp.float32), axis=1, keepdims=True)

    @pl.when(t >= ni)
    def _offdiag():
        tc = t - ni
        first = tc < ni * (ni // 2 - 1)
        i = jnp.where(first, tc % ni, tc - ni * (ni // 2 - 1))
        off = jnp.where(first, tc // ni + 1, ni // 2)
        j = jnp.where(first, (i + off) % ni, i + ni // 2)

        fi = fn8[pl.ds(i * b, b), :]
        fj = fn8[pl.ds(j * b, b), :]
        raw = jax.lax.dot_general(fi, fj, (((1,), (1,)), ((), ())),
                                  preferred_element_type=jnp.float32)

        labj = labj_ref[...]
        ohj = (labj == jax.lax.broadcasted_iota(jnp.int32, (C, b), 0)
               ).astype(jnp.float8_e4m3fn)

        e = jnp.exp2(raw * _EXP_MUL - _EXP_OFF)
        sim8 = (raw * _SIM_SCALE).astype(jnp.float8_e4m3fn)
        sr_acc[pl.ds(i * b, b), :] += jnp.sum(e, axis=1, keepdims=True)
        g_acc[pl.ds(i * b, b), :] += jax.lax.dot_general(
            sim8, ohj, (((1,), (1,)), ((), ())),
            preferred_element_type=jnp.float32)

        sc_acc[:, pl.ds(j * b, b)] += jnp.sum(e, axis=0, keepdims=True)
        labi = labi_ref[...]
        ohi = (labi == jax.lax.broadcasted_iota(jnp.int32, (C, b), 0)
               ).astype(jnp.float8_e4m3fn)
        g_acc[pl.ds(j * b, b), :] += jax.lax.dot_general(
            sim8, ohi, (((0,), (1,)), ((), ())),
            preferred_element_type=jnp.float32)

    @pl.when(t == nt - 1)
    def _emit():
        B = ni * b
        S = sr_acc[...] + jnp.transpose(sc_acc[...])
        sw = sw_ref[...]
        ohi = (labcol_ref[...] == jax.lax.broadcasted_iota(
            jnp.int32, (B, C), 1)).astype(jnp.float32)
        sw_il = jnp.sum(sw * ohi, axis=1, keepdims=True)
        W = jnp.dot(sw, c_acc[...], preferred_element_type=jnp.float32) - sw_il
        P = jnp.sum(sw * g_acc[...], axis=1, keepdims=True)
        logden = _INV_T + jnp.log(S + _EPS)
        loss = -(_TEMP / _BASE_TEMP) * (P - W * logden) / (W + _EPS)
        out_ref[...] = jnp.sum(loss, keepdims=True).reshape(1, 1) * (1.0 / B)


@jax.jit
def kernel(features, labels, similarity_weights):
    B, D = features.shape
    C = similarity_weights.shape[1]
    b = 1024
    ni = B // b
    nfull = ni * (ni // 2 - 1)
    nt = ni + nfull + ni // 2

    lab32 = labels.astype(jnp.int32)
    labrow = lab32.reshape(1, B)
    labcol = lab32.reshape(B, 1)

    def _ij(t):
        tc = jnp.maximum(t - ni, 0)
        first = tc < nfull
        i = jnp.where(first, tc % ni, tc - nfull)
        off = jnp.where(first, tc // ni + 1, ni // 2)
        j = jnp.where(first, (i + off) % ni, i + ni // 2)
        return i, j

    def _i_map(t):
        i, _ = _ij(t)
        return (0, jnp.where(t < ni, t, i))

    def _j_map(t):
        _, j = _ij(t)
        return (0, jnp.where(t < ni, t, j))

    out = pl.pallas_call(
        functools.partial(_wsc_kernel, b=b, ni=ni, nt=nt, C=C),
        grid=(nt,),
        in_specs=[
            pl.BlockSpec((b, D), lambda t: (jnp.minimum(t, ni - 1), 0)),
            pl.BlockSpec((1, b), _i_map),
            pl.BlockSpec((1, b), _j_map),
            pl.BlockSpec((B, C), lambda t: (0, 0)),
            pl.BlockSpec((B, 1), lambda t: (0, 0)),
        ],
        out_specs=pl.BlockSpec((1, 1), lambda t: (0, 0)),
        out_shape=jax.ShapeDtypeStruct((1, 1), jnp.float32),
        scratch_shapes=[
            pltpu.VMEM((B, D), jnp.float8_e4m3fn),
            pltpu.VMEM((B, 1), jnp.float32),
            pltpu.VMEM((1, B), jnp.float32),
            pltpu.VMEM((B, C), jnp.float32),
            pltpu.VMEM((C, 1), jnp.float32),
        ],
        compiler_params=pltpu.CompilerParams(
            dimension_semantics=("arbitrary",)),
    )(features, labrow, labrow, similarity_weights, labcol)
    return out[0, 0]
